# Initial kernel scaffold; baseline (speedup 1.0000x reference)
import functools

import jax
import jax.numpy as jnp
from jax import lax
from jax.experimental import pallas as pl
from jax.experimental.pallas import tpu as pltpu
from jax.experimental.pallas import tpu_sc as plsc

_VOCAB = 1000000
_DIM = 32
_BATCH = 4096
_HIST = 200
_SCALE = float(_DIM) ** 0.5

_NW = 32
_B = _BATCH * _HIST
_PER_W = _B // _NW
_CHUNK = 1600
_NCHUNK = _PER_W // _CHUNK


def _sc_body(ids_hbm, table_hbm, out_hbm, idx_v, rows_v, sem):
    wid = lax.axis_index("s") * 2 + lax.axis_index("c")
    wbase = wid * _PER_W

    def chunk_body(c, _):
        base = wbase + c * _CHUNK
        pltpu.sync_copy(ids_hbm.at[pl.ds(base, _CHUNK)], idx_v)
        pltpu.async_copy(table_hbm.at[idx_v], rows_v, sem).wait()

        def scale_body(i, _):
            rows_v[i, pl.ds(0, 16)] = rows_v[i, pl.ds(0, 16)] * _SCALE
            rows_v[i, pl.ds(16, 16)] = rows_v[i, pl.ds(16, 16)] * _SCALE
            return 0

        lax.fori_loop(0, _CHUNK, scale_body, 0)
        pltpu.sync_copy(rows_v, out_hbm.at[pl.ds(base, _CHUNK)])
        return 0

    lax.fori_loop(0, _NCHUNK, chunk_body, 0)


@jax.jit
def _embed(ids_flat, table):
    mesh = plsc.VectorSubcoreMesh(core_axis_name="c", subcore_axis_name="s")
    run = functools.partial(
        pl.kernel,
        mesh=mesh,
        out_type=jax.ShapeDtypeStruct((_B, _DIM), jnp.float32),
        scratch_types=[
            pltpu.VMEM((_CHUNK,), jnp.int32),
            pltpu.VMEM((_CHUNK, _DIM), jnp.float32),
            pltpu.SemaphoreType.DMA,
        ],
    )(_sc_body)
    return run(ids_flat, table)


def kernel(input_ids, table):
    ids_flat = input_ids.reshape(-1).astype(jnp.int32)
    out = _embed(ids_flat, table)
    return out.reshape(_BATCH, _HIST, _DIM)


# speedup vs baseline: 1.3072x; 1.3072x over previous
import functools

import jax
import jax.numpy as jnp
from jax import lax
from jax.experimental import pallas as pl
from jax.experimental.pallas import tpu as pltpu
from jax.experimental.pallas import tpu_sc as plsc

_VOCAB = 1000000
_DIM = 32
_BATCH = 4096
_HIST = 200
_SCALE = float(_DIM) ** 0.5

_NW = 32
_B = _BATCH * _HIST
_PER_W = _B // _NW
_CHUNK = 1600
_NCHUNK = _PER_W // _CHUNK


def _sc_body(ids_hbm, table_hbm, out_hbm, idx_v, rows_v, sem):
    wid = lax.axis_index("s") * 2 + lax.axis_index("c")
    wbase = wid * _PER_W

    def chunk_body(c, _):
        base = wbase + c * _CHUNK
        pltpu.sync_copy(ids_hbm.at[pl.ds(base, _CHUNK)], idx_v)
        pltpu.async_copy(table_hbm.at[idx_v], rows_v, sem).wait()

        def scale_body(i, _):
            rows_v[i, pl.ds(0, 16)] = rows_v[i, pl.ds(0, 16)] * _SCALE
            rows_v[i, pl.ds(16, 16)] = rows_v[i, pl.ds(16, 16)] * _SCALE
            return 0

        lax.fori_loop(0, _CHUNK, scale_body, 0)
        pltpu.sync_copy(rows_v, out_hbm.at[pl.ds(base, _CHUNK)])
        return 0

    lax.fori_loop(0, _NCHUNK, chunk_body, 0)


@jax.jit
def _embed(ids_flat, table):
    mesh = plsc.VectorSubcoreMesh(core_axis_name="c", subcore_axis_name="s")
    run = functools.partial(
        pl.kernel,
        mesh=mesh,
        out_type=jax.ShapeDtypeStruct((_B, _DIM), jnp.float32),
        scratch_types=[
            pltpu.VMEM((_CHUNK,), jnp.int32),
            pltpu.VMEM((_CHUNK, _DIM), jnp.float32),
            pltpu.SemaphoreType.DMA,
        ],
        compiler_params=pltpu.CompilerParams(use_tc_tiling_on_sc=False),
    )(_sc_body)
    return run(ids_flat, table)


def kernel(input_ids, table):
    ids_flat = input_ids.reshape(-1).astype(jnp.int32)
    out = _embed(ids_flat, table)
    return out.reshape(_BATCH, _HIST, _DIM)


# speedup vs baseline: 1.4790x; 1.1315x over previous
import functools

import jax
import jax.numpy as jnp
from jax import lax
from jax.experimental import pallas as pl
from jax.experimental.pallas import tpu as pltpu
from jax.experimental.pallas import tpu_sc as plsc

_VOCAB = 1000000
_DIM = 32
_BATCH = 4096
_HIST = 200
_SCALE = float(_DIM) ** 0.5

_NW = 32
_B = _BATCH * _HIST
_PER_W = _B // _NW
_CHUNK = 640
_NCHUNK = _PER_W // _CHUNK
_ROUNDS = _NCHUNK // 2
_UNROLL = 8


def _sc_body(ids_hbm, table_hbm, out_hbm,
             idx_all, g0, g1, s0, s1, sg0, sg1, ss0, ss1):
    wid = lax.axis_index("s") * 2 + lax.axis_index("c")
    wbase = wid * _PER_W

    gbuf = (g0, g1)
    sbuf = (s0, s1)
    gsem = (sg0, sg1)
    ssem = (ss0, ss1)

    pltpu.sync_copy(ids_hbm.at[pl.ds(wbase, _PER_W)], idx_all)

    def start_gather(c, b):
        pltpu.async_copy(
            table_hbm.at[idx_all.at[pl.ds(c * _CHUNK, _CHUNK)]],
            gbuf[b], gsem[b])

    def wait_gather(b):
        pltpu.make_async_copy(
            table_hbm.at[idx_all.at[pl.ds(0, _CHUNK)]],
            gbuf[b], gsem[b]).wait()

    def start_scatter(c, b):
        pltpu.async_copy(
            sbuf[b], out_hbm.at[pl.ds(wbase + c * _CHUNK, _CHUNK)], ssem[b])

    def wait_scatter(b):
        pltpu.make_async_copy(
            sbuf[b], out_hbm.at[pl.ds(wbase, _CHUNK)], ssem[b]).wait()

    def scale(b):
        src = gbuf[b]
        dst = sbuf[b]

        def scale_iter(i, _):
            r = i * _UNROLL
            for u in range(_UNROLL):
                dst[r + u, pl.ds(0, 16)] = src[r + u, pl.ds(0, 16)] * _SCALE
                dst[r + u, pl.ds(16, 16)] = src[r + u, pl.ds(16, 16)] * _SCALE
            return 0

        lax.fori_loop(0, _CHUNK // _UNROLL, scale_iter, 0)

    start_gather(0, 0)
    start_gather(1, 1)

    for b in (0, 1):
        wait_gather(b)
        scale(b)
        start_scatter(b, b)
        start_gather(2 + b, b)

    def round_body(r, _):
        for b in (0, 1):
            c = 2 * r + b
            wait_gather(b)
            wait_scatter(b)
            scale(b)
            start_scatter(c, b)

            @pl.when(c + 2 < _NCHUNK)
            def _():
                start_gather(c + 2, b)

        return 0

    lax.fori_loop(1, _ROUNDS, round_body, 0)

    wait_scatter(0)
    wait_scatter(1)


@jax.jit
def _embed(ids_flat, table):
    mesh = plsc.VectorSubcoreMesh(core_axis_name="c", subcore_axis_name="s")
    run = functools.partial(
        pl.kernel,
        mesh=mesh,
        out_type=jax.ShapeDtypeStruct((_B, _DIM), jnp.float32),
        scratch_types=[
            pltpu.VMEM((_PER_W,), jnp.int32),
            pltpu.VMEM((_CHUNK, _DIM), jnp.float32),
            pltpu.VMEM((_CHUNK, _DIM), jnp.float32),
            pltpu.VMEM((_CHUNK, _DIM), jnp.float32),
            pltpu.VMEM((_CHUNK, _DIM), jnp.float32),
            pltpu.SemaphoreType.DMA,
            pltpu.SemaphoreType.DMA,
            pltpu.SemaphoreType.DMA,
            pltpu.SemaphoreType.DMA,
        ],
        compiler_params=pltpu.CompilerParams(use_tc_tiling_on_sc=False),
    )(_sc_body)
    return run(ids_flat, table)


def kernel(input_ids, table):
    ids_flat = input_ids.reshape(-1).astype(jnp.int32)
    out = _embed(ids_flat, table)
    return out.reshape(_BATCH, _HIST, _DIM)
